# baseline (speedup 1.0000x reference)
import functools

import jax
import jax.numpy as jnp
from jax import lax
from jax.experimental import pallas as pl
from jax.experimental.pallas import tpu as pltpu
from jax.experimental.pallas import tpu_sc as plsc

BATCH = 16384
SEQ = 50
DIM = 64
NC = 2
NS = 16
NW = NC * NS
BPW = BATCH // NW
NB = 4
NBUF = 4
NSTEP = BPW // NB
NG = NSTEP // NBUF


def _emb_body(idx_hbm, w_hbm, out_hbm, idx_v, bufs, gsems, ssems):
    wid = lax.axis_index("s") * NC + lax.axis_index("c")
    base = wid * BPW
    pltpu.sync_copy(idx_hbm.at[pl.ds(base, BPW)], idx_v)

    def gather(j, b, q):
        return pltpu.make_async_copy(
            w_hbm.at[idx_v.at[j * NB + q]], bufs[b].at[q], gsems[b])

    def fill(j, b):
        for q in range(NB):
            gather(j, b, q).start()

    def wait_fill(j, b):
        for q in range(NB):
            gather(j, b, q).wait()

    for b in range(NBUF):
        fill(b, b)

    def step(g, _):
        for b in range(NBUF):
            j = g * NBUF + b
            wait_fill(j, b)
            store = pltpu.make_async_copy(
                bufs[b], out_hbm.at[pl.ds(base + j * NB, NB)], ssems[b])
            store.start()

            @pl.when(g < NG - 1)
            def _():
                store.wait()
                fill(j + NBUF, b)

        return 0

    lax.fori_loop(0, NG, step, 0)

    for b in range(NBUF):
        pltpu.make_async_copy(
            bufs[b], out_hbm.at[pl.ds(0, NB)], ssems[b]).wait()


def kernel(token_ids, weight):
    mesh = plsc.VectorSubcoreMesh(core_axis_name="c", subcore_axis_name="s")

    @functools.partial(
        pl.kernel,
        mesh=mesh,
        out_type=jax.ShapeDtypeStruct((BATCH, SEQ, DIM), jnp.float32),
        compiler_params=pltpu.CompilerParams(use_tc_tiling_on_sc=False),
        scratch_types=[
            pltpu.VMEM((BPW, SEQ), jnp.int32),
            *[pltpu.VMEM((NB, SEQ, DIM), jnp.float32) for _ in range(NBUF)],
            *[pltpu.SemaphoreType.DMA for _ in range(2 * NBUF)],
        ],
    )
    def emb(idx_hbm, w_hbm, out_hbm, idx_v, *rest):
        bufs = rest[:NBUF]
        gsems = rest[NBUF:2 * NBUF]
        ssems = rest[2 * NBUF:]
        _emb_body(idx_hbm, w_hbm, out_hbm, idx_v, bufs, gsems, ssems)

    return emb(token_ids, weight)
